# baseline (speedup 1.0000x reference)
import functools

import jax
import jax.numpy as jnp
from jax import lax
from jax.experimental import pallas as pl
from jax.experimental.pallas import tpu as pltpu
from jax.experimental.pallas import tpu_sc as plsc

_N = 10000
_E = 320000
_DIN = 128
_DH = 256
_H = 128
_ALPHA = 0.5

_NS = 16
_C = 128
_CH = 158
_EPAD = _NS * _CH * _C
_NPAD = 10112
_RPT = _NPAD // _NS
_LAST = _N - (_NS - 1) * _RPT

_R = 1000


def _ln_tanh(v, gamma, beta):
    mu = jnp.mean(v, axis=-1, keepdims=True)
    d = v - mu
    var = jnp.mean(d * d, axis=-1, keepdims=True)
    return jnp.tanh(d * lax.rsqrt(var + 1e-5) * gamma + beta)


def _pre_body(x, We, be, W2, b1, gamma, beta, nf, xh_o, cc_o, z_o, u0_o, u1_o):
    xh = jnp.dot(x[...], We[...], preferred_element_type=jnp.float32) + be[...]
    cc = jnp.dot(xh, W2[...], preferred_element_type=jnp.float32) + b1[...]
    h = _ln_tanh(cc, gamma[...], beta[...])
    z = _ALPHA * h
    u = z * nf[...]
    xh_o[...] = xh
    cc_o[...] = cc
    z_o[...] = z
    u0_o[...] = u[:, :_H]
    u1_o[...] = u[:, _H:]


def _step_body(agg0, agg1, z, cc, nf, xh, W1a, W1b, gamma, beta,
               z_o, u0_o, u1_o, out_o):
    nfv = nf[...]
    a0 = agg0[...] * nfv
    a1 = agg1[...] * nfv
    hin = (jnp.dot(a0, W1a[...], preferred_element_type=jnp.float32)
           + jnp.dot(a1, W1b[...], preferred_element_type=jnp.float32)
           + cc[...])
    h = _ln_tanh(hin, gamma[...], beta[...])
    zn = (1.0 - _ALPHA) * z[...] + _ALPHA * h
    z_o[...] = zn
    u0_o[...] = zn[:, :_H] * nfv
    u1_o[...] = zn[:, _H:] * nfv
    out_o[...] = nfv * zn + xh[...]


def _row_spec(cols):
    return pl.BlockSpec((_R, cols), lambda i: (i, 0))


def _const_spec(shape):
    return pl.BlockSpec(shape, lambda i: (0, 0))


_GRID = (_N // _R,)
_f32 = jnp.float32

_pre_call = pl.pallas_call(
    _pre_body,
    grid=_GRID,
    in_specs=[
        _row_spec(_DIN),
        _const_spec((_DIN, _DH)),
        _const_spec((1, _DH)),
        _const_spec((_DH, _DH)),
        _const_spec((1, _DH)),
        _const_spec((1, _DH)),
        _const_spec((1, _DH)),
        _row_spec(1),
    ],
    out_specs=[_row_spec(_DH), _row_spec(_DH), _row_spec(_DH),
               _row_spec(_H), _row_spec(_H)],
    out_shape=[jax.ShapeDtypeStruct((_N, _DH), _f32),
               jax.ShapeDtypeStruct((_N, _DH), _f32),
               jax.ShapeDtypeStruct((_N, _DH), _f32),
               jax.ShapeDtypeStruct((_N, _H), _f32),
               jax.ShapeDtypeStruct((_N, _H), _f32)],
)

_step_call = pl.pallas_call(
    _step_body,
    grid=_GRID,
    in_specs=[
        _row_spec(_H),
        _row_spec(_H),
        _row_spec(_DH),
        _row_spec(_DH),
        _row_spec(1),
        _row_spec(_DH),
        _const_spec((_H, _DH)),
        _const_spec((_H, _DH)),
        _const_spec((1, _DH)),
        _const_spec((1, _DH)),
    ],
    out_specs=[_row_spec(_DH), _row_spec(_H), _row_spec(_H), _row_spec(_DH)],
    out_shape=[jax.ShapeDtypeStruct((_N, _DH), _f32),
               jax.ShapeDtypeStruct((_N, _H), _f32),
               jax.ShapeDtypeStruct((_N, _H), _f32),
               jax.ShapeDtypeStruct((_N, _DH), _f32)],
)


def _spmm_body(srcp, dstp, u0, u1, zeros, out0, out1,
               src_idx, dst_idx, buf, acc, gsem, isem, ssem):
    c = lax.axis_index("c")
    s = lax.axis_index("s")
    pltpu.sync_copy(zeros, acc.at[pl.ds(s * _RPT, _RPT)])
    plsc.subcore_barrier()

    def run(u_hbm):
        def idx_start(j, slot):
            pltpu.async_copy(srcp.at[s, j], src_idx.at[slot], isem)
            pltpu.async_copy(dstp.at[s, j], dst_idx.at[slot], isem)

        def idx_wait(j, slot):
            pltpu.make_async_copy(srcp.at[s, j], src_idx.at[slot], isem).wait()
            pltpu.make_async_copy(dstp.at[s, j], dst_idx.at[slot], isem).wait()

        def gat_start(slot):
            pltpu.async_copy(u_hbm.at[src_idx.at[slot]], buf.at[slot], gsem)

        def gat_wait(slot):
            pltpu.make_async_copy(u_hbm.at[src_idx.at[slot]],
                                  buf.at[slot], gsem).wait()

        def scat_start(slot):
            pltpu.async_copy(buf.at[slot], acc.at[dst_idx.at[slot]], ssem,
                             add=True)

        def scat_wait(slot):
            pltpu.make_async_copy(buf.at[slot], acc.at[dst_idx.at[slot]],
                                  ssem).wait()

        pltpu.sync_copy(srcp.at[s, 0], src_idx.at[0])
        pltpu.sync_copy(dstp.at[s, 0], dst_idx.at[0])
        gat_start(0)
        idx_start(1, 1)

        def pair(p, carry):
            j0 = 2 * p
            j1 = j0 + 1
            gat_wait(0)
            idx_wait(j1, 1)
            gat_start(1)
            scat_start(0)
            gat_wait(1)
            scat_wait(0)
            pl.when(j0 + 2 < _CH)(lambda: idx_start(j0 + 2, 0))
            scat_start(1)

            def cont0():
                idx_wait(j0 + 2, 0)
                gat_start(0)
            pl.when(j0 + 2 < _CH)(cont0)
            scat_wait(1)
            pl.when(j1 + 2 < _CH)(lambda: idx_start(j1 + 2, 1))
            return carry

        lax.fori_loop(0, _CH // 2, pair, 0)

    pl.when(c == 0)(lambda: run(u0))
    pl.when(c == 1)(lambda: run(u1))
    plsc.subcore_barrier()

    def drain(out_hbm):
        def full():
            pltpu.sync_copy(acc.at[pl.ds(s * _RPT, _RPT)],
                            out_hbm.at[pl.ds(s * _RPT, _RPT)])
        def part():
            pltpu.sync_copy(acc.at[pl.ds(s * _RPT, _LAST)],
                            out_hbm.at[pl.ds(s * _RPT, _LAST)])
        pl.when(s < _NS - 1)(full)
        pl.when(s == _NS - 1)(part)

    pl.when(c == 0)(lambda: drain(out0))
    pl.when(c == 1)(lambda: drain(out1))


_spmm_call = pl.kernel(
    _spmm_body,
    out_type=[jax.ShapeDtypeStruct((_N, _H), _f32),
              jax.ShapeDtypeStruct((_N, _H), _f32)],
    mesh=plsc.VectorSubcoreMesh(core_axis_name="c", subcore_axis_name="s"),
    scratch_types=[
        pltpu.VMEM((2, _C), jnp.int32),
        pltpu.VMEM((2, _C), jnp.int32),
        pltpu.VMEM((2, _C, _H), _f32),
        pltpu.VMEM_SHARED((_NPAD, _H), _f32),
        pltpu.SemaphoreType.DMA,
        pltpu.SemaphoreType.DMA,
        pltpu.SemaphoreType.DMA,
    ],
)


def kernel(x, edge_index, norm_factor, We, be, W1, W2, b1, gamma, beta):
    src = edge_index[0]
    dst = edge_index[1]
    pad = _EPAD - _E
    pad_src = (jnp.arange(pad, dtype=jnp.int32) * 2789) % _N
    pad_dst = _N + (jnp.arange(pad, dtype=jnp.int32) % (_NPAD - _N))
    srcp = jnp.concatenate([src, pad_src]).reshape(_NS, _CH, _C)
    dstp = jnp.concatenate([dst, pad_dst]).reshape(_NS, _CH, _C)
    zeros = jnp.zeros((_RPT, _H), _f32)

    be2 = be.reshape(1, _DH)
    b12 = b1.reshape(1, _DH)
    g2 = gamma.reshape(1, _DH)
    bt2 = beta.reshape(1, _DH)
    W1a = W1[:_H]
    W1b = W1[_H:]

    xh, cc, z, u0, u1 = _pre_call(x, We, be2, W2, b12, g2, bt2, norm_factor)
    out = None
    for _ in range(3):
        agg0, agg1 = _spmm_call(srcp, dstp, u0, u1, zeros)
        z, u0, u1, out = _step_call(agg0, agg1, z, cc, norm_factor, xh,
                                    W1a, W1b, g2, bt2)
    return out


# speedup vs baseline: 1.1222x; 1.1222x over previous
import functools

import jax
import jax.numpy as jnp
from jax import lax
from jax.experimental import pallas as pl
from jax.experimental.pallas import tpu as pltpu
from jax.experimental.pallas import tpu_sc as plsc

_N = 10000
_E = 320000
_DIN = 128
_DH = 256
_H = 128
_ALPHA = 0.5

_NS = 16
_C = 128
_CH = 158
_EPAD = _NS * _CH * _C
_NPAD = 10112
_RPT = _NPAD // _NS
_LAST = _N - (_NS - 1) * _RPT

_R = 1000


def _ln_tanh(v, gamma, beta):
    mu = jnp.mean(v, axis=-1, keepdims=True)
    d = v - mu
    var = jnp.mean(d * d, axis=-1, keepdims=True)
    return jnp.tanh(d * lax.rsqrt(var + 1e-5) * gamma + beta)


def _pre_body(x, We, be, W2, b1, gamma, beta, nf, xh_o, cc_o, z_o, u0_o, u1_o):
    xh = jnp.dot(x[...], We[...], preferred_element_type=jnp.float32) + be[...]
    cc = jnp.dot(xh, W2[...], preferred_element_type=jnp.float32) + b1[...]
    h = _ln_tanh(cc, gamma[...], beta[...])
    z = _ALPHA * h
    u = z * nf[...]
    xh_o[...] = xh
    cc_o[...] = cc
    z_o[...] = z
    u0_o[...] = u[:, :_H]
    u1_o[...] = u[:, _H:]


def _step_body(agg0, agg1, z, cc, nf, xh, W1a, W1b, gamma, beta,
               z_o, u0_o, u1_o, out_o):
    nfv = nf[...]
    a0 = agg0[...] * nfv
    a1 = agg1[...] * nfv
    hin = (jnp.dot(a0, W1a[...], preferred_element_type=jnp.float32)
           + jnp.dot(a1, W1b[...], preferred_element_type=jnp.float32)
           + cc[...])
    h = _ln_tanh(hin, gamma[...], beta[...])
    zn = (1.0 - _ALPHA) * z[...] + _ALPHA * h
    z_o[...] = zn
    u0_o[...] = zn[:, :_H] * nfv
    u1_o[...] = zn[:, _H:] * nfv
    out_o[...] = nfv * zn + xh[...]


def _row_spec(cols):
    return pl.BlockSpec((_R, cols), lambda i: (i, 0))


def _const_spec(shape):
    return pl.BlockSpec(shape, lambda i: (0, 0))


_GRID = (_N // _R,)
_f32 = jnp.float32

_pre_call = pl.pallas_call(
    _pre_body,
    grid=_GRID,
    in_specs=[
        _row_spec(_DIN),
        _const_spec((_DIN, _DH)),
        _const_spec((1, _DH)),
        _const_spec((_DH, _DH)),
        _const_spec((1, _DH)),
        _const_spec((1, _DH)),
        _const_spec((1, _DH)),
        _row_spec(1),
    ],
    out_specs=[_row_spec(_DH), _row_spec(_DH), _row_spec(_DH),
               _row_spec(_H), _row_spec(_H)],
    out_shape=[jax.ShapeDtypeStruct((_N, _DH), _f32),
               jax.ShapeDtypeStruct((_N, _DH), _f32),
               jax.ShapeDtypeStruct((_N, _DH), _f32),
               jax.ShapeDtypeStruct((_N, _H), _f32),
               jax.ShapeDtypeStruct((_N, _H), _f32)],
)

_step_call = pl.pallas_call(
    _step_body,
    grid=_GRID,
    in_specs=[
        _row_spec(_H),
        _row_spec(_H),
        _row_spec(_DH),
        _row_spec(_DH),
        _row_spec(1),
        _row_spec(_DH),
        _const_spec((_H, _DH)),
        _const_spec((_H, _DH)),
        _const_spec((1, _DH)),
        _const_spec((1, _DH)),
    ],
    out_specs=[_row_spec(_DH), _row_spec(_H), _row_spec(_H), _row_spec(_DH)],
    out_shape=[jax.ShapeDtypeStruct((_N, _DH), _f32),
               jax.ShapeDtypeStruct((_N, _H), _f32),
               jax.ShapeDtypeStruct((_N, _H), _f32),
               jax.ShapeDtypeStruct((_N, _DH), _f32)],
)


def _spmm_body(srcp, dstp, u0, u1, zeros, out0, out1,
               src_idx, dst_idx, buf, acc, gsem, isem):
    c = lax.axis_index("c")
    s = lax.axis_index("s")
    pltpu.sync_copy(zeros, acc.at[pl.ds(s * _RPT, _RPT)])
    plsc.subcore_barrier()

    def run(u_hbm):
        def idx_start(j, slot):
            pltpu.async_copy(srcp.at[s, j], src_idx.at[slot], isem)
            pltpu.async_copy(dstp.at[s, j], dst_idx.at[slot], isem)

        def idx_wait(slot):
            pltpu.make_async_copy(srcp.at[s, 0], src_idx.at[slot], isem).wait()
            pltpu.make_async_copy(dstp.at[s, 0], dst_idx.at[slot], isem).wait()

        def gat_start(slot):
            pltpu.async_copy(u_hbm.at[src_idx.at[slot]], buf.at[slot], gsem)

        def gat_wait(slot):
            pltpu.make_async_copy(u_hbm.at[pl.ds(0, _C)], buf.at[slot],
                                  gsem).wait()

        def scat(slot):
            pltpu.sync_copy(buf.at[slot], acc.at[dst_idx.at[slot]], add=True)

        pltpu.sync_copy(srcp.at[s, 0], src_idx.at[0])
        pltpu.sync_copy(dstp.at[s, 0], dst_idx.at[0])
        gat_start(0)
        idx_start(1, 1)

        def pair(p, carry):
            j0 = 2 * p
            j1 = j0 + 1
            gat_wait(0)
            idx_wait(1)
            gat_start(1)
            scat(0)
            pl.when(j0 + 2 < _CH)(lambda: idx_start(j0 + 2, 0))
            gat_wait(1)

            def cont():
                idx_wait(0)
                gat_start(0)
            pl.when(j0 + 2 < _CH)(cont)
            scat(1)
            pl.when(j1 + 2 < _CH)(lambda: idx_start(j1 + 2, 1))
            return carry

        lax.fori_loop(0, _CH // 2, pair, 0)

    pl.when(c == 0)(lambda: run(u0))
    pl.when(c == 1)(lambda: run(u1))
    plsc.subcore_barrier()

    def drain(out_hbm):
        def full():
            pltpu.sync_copy(acc.at[pl.ds(s * _RPT, _RPT)],
                            out_hbm.at[pl.ds(s * _RPT, _RPT)])
        def part():
            pltpu.sync_copy(acc.at[pl.ds(s * _RPT, _LAST)],
                            out_hbm.at[pl.ds(s * _RPT, _LAST)])
        pl.when(s < _NS - 1)(full)
        pl.when(s == _NS - 1)(part)

    pl.when(c == 0)(lambda: drain(out0))
    pl.when(c == 1)(lambda: drain(out1))


_spmm_call = pl.kernel(
    _spmm_body,
    out_type=[jax.ShapeDtypeStruct((_N, _H), _f32),
              jax.ShapeDtypeStruct((_N, _H), _f32)],
    mesh=plsc.VectorSubcoreMesh(core_axis_name="c", subcore_axis_name="s"),
    scratch_types=[
        pltpu.VMEM((2, _C), jnp.int32),
        pltpu.VMEM((2, _C), jnp.int32),
        pltpu.VMEM((2, _C, _H), _f32),
        pltpu.VMEM_SHARED((_NPAD, _H), _f32),
        pltpu.SemaphoreType.DMA,
        pltpu.SemaphoreType.DMA,
    ],
)


def kernel(x, edge_index, norm_factor, We, be, W1, W2, b1, gamma, beta):
    src = edge_index[0]
    dst = edge_index[1]
    pad = _EPAD - _E
    pad_src = (jnp.arange(pad, dtype=jnp.int32) * 2789) % _N
    pad_dst = _N + (jnp.arange(pad, dtype=jnp.int32) % (_NPAD - _N))
    srcp = jnp.concatenate([src, pad_src]).reshape(_NS, _CH, _C)
    dstp = jnp.concatenate([dst, pad_dst]).reshape(_NS, _CH, _C)
    zeros = jnp.zeros((_RPT, _H), _f32)

    be2 = be.reshape(1, _DH)
    b12 = b1.reshape(1, _DH)
    g2 = gamma.reshape(1, _DH)
    bt2 = beta.reshape(1, _DH)
    W1a = W1[:_H]
    W1b = W1[_H:]

    xh, cc, z, u0, u1 = _pre_call(x, We, be2, W2, b12, g2, bt2, norm_factor)
    out = None
    for _ in range(3):
        agg0, agg1 = _spmm_call(srcp, dstp, u0, u1, zeros)
        z, u0, u1, out = _step_call(agg0, agg1, z, cc, norm_factor, xh,
                                    W1a, W1b, g2, bt2)
    return out


# speedup vs baseline: 1.1338x; 1.0104x over previous
import functools

import jax
import jax.numpy as jnp
from jax import lax
from jax.experimental import pallas as pl
from jax.experimental.pallas import tpu as pltpu
from jax.experimental.pallas import tpu_sc as plsc

_N = 10000
_E = 320000
_DIN = 128
_DH = 256
_H = 128
_ALPHA = 0.5

_NS = 16
_C = 128
_CH = 158
_EPAD = _NS * _CH * _C
_NPAD = 10112
_RPT = _NPAD // _NS
_LAST = _N - (_NS - 1) * _RPT

_R = 1000


def _ln_tanh(v, gamma, beta):
    mu = jnp.mean(v, axis=-1, keepdims=True)
    d = v - mu
    var = jnp.mean(d * d, axis=-1, keepdims=True)
    return jnp.tanh(d * lax.rsqrt(var + 1e-5) * gamma + beta)


def _pre_body(x, We, be, W2, b1, gamma, beta, nf, xh_o, cc_o, u0_o, u1_o):
    xh = jnp.dot(x[...], We[...], preferred_element_type=jnp.float32) + be[...]
    cc = jnp.dot(xh, W2[...], preferred_element_type=jnp.float32) + b1[...]
    h = _ln_tanh(cc, gamma[...], beta[...])
    u = (_ALPHA * h) * nf[...]
    xh_o[...] = xh
    cc_o[...] = cc
    u0_o[...] = u[:, :_H]
    u1_o[...] = u[:, _H:]


def _step_h(agg0, agg1, cc, nfv, W1a, W1b, gamma, beta):
    a0 = agg0[...] * nfv
    a1 = agg1[...] * nfv
    hin = (jnp.dot(a0, W1a[...], preferred_element_type=jnp.float32)
           + jnp.dot(a1, W1b[...], preferred_element_type=jnp.float32)
           + cc[...])
    return _ln_tanh(hin, gamma[...], beta[...])


def _mid_body(agg0, agg1, u0, u1, cc, nf, W1a, W1b, gamma, beta, u0_o, u1_o):
    nfv = nf[...]
    h = _step_h(agg0, agg1, cc, nfv, W1a, W1b, gamma, beta)
    u0_o[...] = (1.0 - _ALPHA) * u0[...] + (_ALPHA * nfv) * h[:, :_H]
    u1_o[...] = (1.0 - _ALPHA) * u1[...] + (_ALPHA * nfv) * h[:, _H:]


def _last_body(agg0, agg1, u0, u1, cc, nf, xh, W1a, W1b, gamma, beta, out_o):
    nfv = nf[...]
    h = _step_h(agg0, agg1, cc, nfv, W1a, W1b, gamma, beta)
    u4 = ((1.0 - _ALPHA) * jnp.concatenate([u0[...], u1[...]], axis=1)
          + (_ALPHA * nfv) * h)
    out_o[...] = u4 + xh[...]


def _row_spec(cols):
    return pl.BlockSpec((_R, cols), lambda i: (i, 0))


def _const_spec(shape):
    return pl.BlockSpec(shape, lambda i: (0, 0))


_GRID = (_N // _R,)
_f32 = jnp.float32

_pre_call = pl.pallas_call(
    _pre_body,
    grid=_GRID,
    in_specs=[
        _row_spec(_DIN),
        _const_spec((_DIN, _DH)),
        _const_spec((1, _DH)),
        _const_spec((_DH, _DH)),
        _const_spec((1, _DH)),
        _const_spec((1, _DH)),
        _const_spec((1, _DH)),
        _row_spec(1),
    ],
    out_specs=[_row_spec(_DH), _row_spec(_DH), _row_spec(_H), _row_spec(_H)],
    out_shape=[jax.ShapeDtypeStruct((_N, _DH), _f32),
               jax.ShapeDtypeStruct((_N, _DH), _f32),
               jax.ShapeDtypeStruct((_N, _H), _f32),
               jax.ShapeDtypeStruct((_N, _H), _f32)],
)

_mid_specs = [
    _row_spec(_H),
    _row_spec(_H),
    _row_spec(_H),
    _row_spec(_H),
    _row_spec(_DH),
    _row_spec(1),
    _const_spec((_H, _DH)),
    _const_spec((_H, _DH)),
    _const_spec((1, _DH)),
    _const_spec((1, _DH)),
]

_mid_call = pl.pallas_call(
    _mid_body,
    grid=_GRID,
    in_specs=_mid_specs,
    out_specs=[_row_spec(_H), _row_spec(_H)],
    out_shape=[jax.ShapeDtypeStruct((_N, _H), _f32),
               jax.ShapeDtypeStruct((_N, _H), _f32)],
)

_last_call = pl.pallas_call(
    _last_body,
    grid=_GRID,
    in_specs=_mid_specs[:6] + [_row_spec(_DH)] + _mid_specs[6:],
    out_specs=[_row_spec(_DH)],
    out_shape=[jax.ShapeDtypeStruct((_N, _DH), _f32)],
)


def _spmm_body(srcp, dstp, u0, u1, zeros, out0, out1,
               src_idx, dst_idx, buf, acc, gsem, isem):
    c = lax.axis_index("c")
    s = lax.axis_index("s")
    pltpu.sync_copy(zeros, acc.at[pl.ds(s * _RPT, _RPT)])
    plsc.subcore_barrier()

    def run(u_hbm):
        def idx_start(j, slot):
            pltpu.async_copy(srcp.at[s, j], src_idx.at[slot], isem)
            pltpu.async_copy(dstp.at[s, j], dst_idx.at[slot], isem)

        def idx_wait(slot):
            pltpu.make_async_copy(srcp.at[s, 0], src_idx.at[slot], isem).wait()
            pltpu.make_async_copy(dstp.at[s, 0], dst_idx.at[slot], isem).wait()

        def gat_start(slot):
            pltpu.async_copy(u_hbm.at[src_idx.at[slot]], buf.at[slot], gsem)

        def gat_wait(slot):
            pltpu.make_async_copy(u_hbm.at[pl.ds(0, _C)], buf.at[slot],
                                  gsem).wait()

        def scat(slot):
            pltpu.sync_copy(buf.at[slot], acc.at[dst_idx.at[slot]], add=True)

        pltpu.sync_copy(srcp.at[s, 0], src_idx.at[0])
        pltpu.sync_copy(dstp.at[s, 0], dst_idx.at[0])
        gat_start(0)
        idx_start(1, 1)

        def pair(p, carry):
            j0 = 2 * p
            j1 = j0 + 1
            gat_wait(0)
            idx_wait(1)
            gat_start(1)
            scat(0)
            pl.when(j0 + 2 < _CH)(lambda: idx_start(j0 + 2, 0))
            gat_wait(1)

            def cont():
                idx_wait(0)
                gat_start(0)
            pl.when(j0 + 2 < _CH)(cont)
            scat(1)
            pl.when(j1 + 2 < _CH)(lambda: idx_start(j1 + 2, 1))
            return carry

        lax.fori_loop(0, _CH // 2, pair, 0)

    pl.when(c == 0)(lambda: run(u0))
    pl.when(c == 1)(lambda: run(u1))
    plsc.subcore_barrier()

    def drain(out_hbm):
        def full():
            pltpu.sync_copy(acc.at[pl.ds(s * _RPT, _RPT)],
                            out_hbm.at[pl.ds(s * _RPT, _RPT)])
        def part():
            pltpu.sync_copy(acc.at[pl.ds(s * _RPT, _LAST)],
                            out_hbm.at[pl.ds(s * _RPT, _LAST)])
        pl.when(s < _NS - 1)(full)
        pl.when(s == _NS - 1)(part)

    pl.when(c == 0)(lambda: drain(out0))
    pl.when(c == 1)(lambda: drain(out1))


_spmm_call = pl.kernel(
    _spmm_body,
    out_type=[jax.ShapeDtypeStruct((_N, _H), _f32),
              jax.ShapeDtypeStruct((_N, _H), _f32)],
    mesh=plsc.VectorSubcoreMesh(core_axis_name="c", subcore_axis_name="s"),
    scratch_types=[
        pltpu.VMEM((2, _C), jnp.int32),
        pltpu.VMEM((2, _C), jnp.int32),
        pltpu.VMEM((2, _C, _H), _f32),
        pltpu.VMEM_SHARED((_NPAD, _H), _f32),
        pltpu.SemaphoreType.DMA,
        pltpu.SemaphoreType.DMA,
    ],
)


def kernel(x, edge_index, norm_factor, We, be, W1, W2, b1, gamma, beta):
    src = edge_index[0]
    dst = edge_index[1]
    pad = _EPAD - _E
    pad_src = (jnp.arange(pad, dtype=jnp.int32) * 2789) % _N
    pad_dst = _N + (jnp.arange(pad, dtype=jnp.int32) % (_NPAD - _N))
    srcp = jnp.concatenate([src, pad_src]).reshape(_NS, _CH, _C)
    dstp = jnp.concatenate([dst, pad_dst]).reshape(_NS, _CH, _C)
    zeros = jnp.zeros((_RPT, _H), _f32)

    be2 = be.reshape(1, _DH)
    b12 = b1.reshape(1, _DH)
    g2 = gamma.reshape(1, _DH)
    bt2 = beta.reshape(1, _DH)
    W1a = W1[:_H]
    W1b = W1[_H:]

    xh, cc, u0, u1 = _pre_call(x, We, be2, W2, b12, g2, bt2, norm_factor)
    for _ in range(2):
        agg0, agg1 = _spmm_call(srcp, dstp, u0, u1, zeros)
        u0, u1 = _mid_call(agg0, agg1, u0, u1, cc, norm_factor,
                           W1a, W1b, g2, bt2)
    agg0, agg1 = _spmm_call(srcp, dstp, u0, u1, zeros)
    out, = _last_call(agg0, agg1, u0, u1, cc, norm_factor, xh,
                      W1a, W1b, g2, bt2)
    return out


# speedup vs baseline: 1.2361x; 1.0902x over previous
import functools

import jax
import jax.numpy as jnp
from jax import lax
from jax.experimental import pallas as pl
from jax.experimental.pallas import tpu as pltpu
from jax.experimental.pallas import tpu_sc as plsc

_N = 10000
_E = 320000
_DIN = 128
_DH = 256
_H = 128
_ALPHA = 0.5

_NS = 16
_C = 128
_CH = 159
_EPAD = _NS * _CH * _C
_NPAD = 10112
_RPT = _NPAD // _NS
_LAST = _N - (_NS - 1) * _RPT

_R = 1000


def _ln_tanh(v, gamma, beta):
    mu = jnp.mean(v, axis=-1, keepdims=True)
    d = v - mu
    var = jnp.mean(d * d, axis=-1, keepdims=True)
    return jnp.tanh(d * lax.rsqrt(var + 1e-5) * gamma + beta)


def _pre_body(x, We, be, W2, b1, gamma, beta, nf, xh_o, cc_o, u0_o, u1_o):
    xh = jnp.dot(x[...], We[...], preferred_element_type=jnp.float32) + be[...]
    cc = jnp.dot(xh, W2[...], preferred_element_type=jnp.float32) + b1[...]
    h = _ln_tanh(cc, gamma[...], beta[...])
    u = (_ALPHA * h) * nf[...]
    xh_o[...] = xh
    cc_o[...] = cc
    u0_o[...] = u[:, :_H]
    u1_o[...] = u[:, _H:]


def _step_h(agg0, agg1, cc, nfv, W1a, W1b, gamma, beta):
    a0 = agg0[...] * nfv
    a1 = agg1[...] * nfv
    hin = (jnp.dot(a0, W1a[...], preferred_element_type=jnp.float32)
           + jnp.dot(a1, W1b[...], preferred_element_type=jnp.float32)
           + cc[...])
    return _ln_tanh(hin, gamma[...], beta[...])


def _mid_body(agg0, agg1, u0, u1, cc, nf, W1a, W1b, gamma, beta, u0_o, u1_o):
    nfv = nf[...]
    h = _step_h(agg0, agg1, cc, nfv, W1a, W1b, gamma, beta)
    u0_o[...] = (1.0 - _ALPHA) * u0[...] + (_ALPHA * nfv) * h[:, :_H]
    u1_o[...] = (1.0 - _ALPHA) * u1[...] + (_ALPHA * nfv) * h[:, _H:]


def _last_body(agg0, agg1, u0, u1, cc, nf, xh, W1a, W1b, gamma, beta, out_o):
    nfv = nf[...]
    h = _step_h(agg0, agg1, cc, nfv, W1a, W1b, gamma, beta)
    u4 = ((1.0 - _ALPHA) * jnp.concatenate([u0[...], u1[...]], axis=1)
          + (_ALPHA * nfv) * h)
    out_o[...] = u4 + xh[...]


def _row_spec(cols):
    return pl.BlockSpec((_R, cols), lambda i: (i, 0))


def _const_spec(shape):
    return pl.BlockSpec(shape, lambda i: (0, 0))


_GRID = (_N // _R,)
_f32 = jnp.float32

_pre_call = pl.pallas_call(
    _pre_body,
    grid=_GRID,
    in_specs=[
        _row_spec(_DIN),
        _const_spec((_DIN, _DH)),
        _const_spec((1, _DH)),
        _const_spec((_DH, _DH)),
        _const_spec((1, _DH)),
        _const_spec((1, _DH)),
        _const_spec((1, _DH)),
        _row_spec(1),
    ],
    out_specs=[_row_spec(_DH), _row_spec(_DH), _row_spec(_H), _row_spec(_H)],
    out_shape=[jax.ShapeDtypeStruct((_N, _DH), _f32),
               jax.ShapeDtypeStruct((_N, _DH), _f32),
               jax.ShapeDtypeStruct((_N, _H), _f32),
               jax.ShapeDtypeStruct((_N, _H), _f32)],
)

_mid_specs = [
    _row_spec(_H),
    _row_spec(_H),
    _row_spec(_H),
    _row_spec(_H),
    _row_spec(_DH),
    _row_spec(1),
    _const_spec((_H, _DH)),
    _const_spec((_H, _DH)),
    _const_spec((1, _DH)),
    _const_spec((1, _DH)),
]

_mid_call = pl.pallas_call(
    _mid_body,
    grid=_GRID,
    in_specs=_mid_specs,
    out_specs=[_row_spec(_H), _row_spec(_H)],
    out_shape=[jax.ShapeDtypeStruct((_N, _H), _f32),
               jax.ShapeDtypeStruct((_N, _H), _f32)],
)

_last_call = pl.pallas_call(
    _last_body,
    grid=_GRID,
    in_specs=_mid_specs[:6] + [_row_spec(_DH)] + _mid_specs[6:],
    out_specs=[_row_spec(_DH)],
    out_shape=[jax.ShapeDtypeStruct((_N, _DH), _f32)],
)


def _spmm_body(srcp, dstp, u0, u1, zeros, out0, out1,
               src_idx, dst_idx, buf, acc, gsem, isem):
    c = lax.axis_index("c")
    s = lax.axis_index("s")

    def run(u_hbm):
        def idx_start(j, slot):
            pltpu.async_copy(srcp.at[s, j], src_idx.at[slot], isem)
            pltpu.async_copy(dstp.at[s, j], dst_idx.at[slot], isem)

        def idx_wait(slot):
            pltpu.make_async_copy(srcp.at[s, 0], src_idx.at[slot], isem).wait()
            pltpu.make_async_copy(dstp.at[s, 0], dst_idx.at[slot], isem).wait()

        def gat_start(slot):
            pltpu.async_copy(u_hbm.at[src_idx.at[slot]], buf.at[slot], gsem)

        def gat_wait(slot):
            pltpu.make_async_copy(u_hbm.at[pl.ds(0, _C)], buf.at[slot],
                                  gsem).wait()

        def scat(slot):
            pltpu.sync_copy(buf.at[slot], acc.at[dst_idx.at[slot]], add=True)

        pltpu.sync_copy(srcp.at[s, 0], src_idx.at[0])
        pltpu.sync_copy(dstp.at[s, 0], dst_idx.at[0])
        pltpu.sync_copy(srcp.at[s, 1], src_idx.at[1])
        pltpu.sync_copy(dstp.at[s, 1], dst_idx.at[1])
        gat_start(0)
        gat_start(1)
        idx_start(2, 2)
        pltpu.sync_copy(zeros, acc.at[pl.ds(s * _RPT, _RPT)])
        plsc.subcore_barrier()

        def triple(q, carry):
            j = 3 * q
            for r in range(3):
                jr = j + r
                b = r
                nb = (r + 2) % 3
                gat_wait(b)

                def nxt(nb=nb):
                    idx_wait(nb)
                    gat_start(nb)
                pl.when(jr + 2 < _CH)(nxt)
                scat(b)
                pl.when(jr + 3 < _CH)(lambda jr=jr, b=b: idx_start(jr + 3, b))
            return carry

        lax.fori_loop(0, _CH // 3, triple, 0)

    pl.when(c == 0)(lambda: run(u0))
    pl.when(c == 1)(lambda: run(u1))
    plsc.subcore_barrier()

    def drain(out_hbm):
        def full():
            pltpu.sync_copy(acc.at[pl.ds(s * _RPT, _RPT)],
                            out_hbm.at[pl.ds(s * _RPT, _RPT)])
        def part():
            pltpu.sync_copy(acc.at[pl.ds(s * _RPT, _LAST)],
                            out_hbm.at[pl.ds(s * _RPT, _LAST)])
        pl.when(s < _NS - 1)(full)
        pl.when(s == _NS - 1)(part)

    pl.when(c == 0)(lambda: drain(out0))
    pl.when(c == 1)(lambda: drain(out1))


_spmm_call = pl.kernel(
    _spmm_body,
    out_type=[jax.ShapeDtypeStruct((_N, _H), _f32),
              jax.ShapeDtypeStruct((_N, _H), _f32)],
    mesh=plsc.VectorSubcoreMesh(core_axis_name="c", subcore_axis_name="s"),
    scratch_types=[
        pltpu.VMEM((3, _C), jnp.int32),
        pltpu.VMEM((3, _C), jnp.int32),
        pltpu.VMEM((3, _C, _H), _f32),
        pltpu.VMEM_SHARED((_NPAD, _H), _f32),
        pltpu.SemaphoreType.DMA,
        pltpu.SemaphoreType.DMA,
    ],
)


def kernel(x, edge_index, norm_factor, We, be, W1, W2, b1, gamma, beta):
    src = edge_index[0]
    dst = edge_index[1]
    pad = _EPAD - _E
    pad_src = (jnp.arange(pad, dtype=jnp.int32) * 2789) % _N
    pad_dst = _N + (jnp.arange(pad, dtype=jnp.int32) % (_NPAD - _N))
    srcp = jnp.concatenate([src, pad_src]).reshape(_NS, _CH, _C)
    dstp = jnp.concatenate([dst, pad_dst]).reshape(_NS, _CH, _C)
    zeros = jnp.zeros((_RPT, _H), _f32)

    be2 = be.reshape(1, _DH)
    b12 = b1.reshape(1, _DH)
    g2 = gamma.reshape(1, _DH)
    bt2 = beta.reshape(1, _DH)
    W1a = W1[:_H]
    W1b = W1[_H:]

    xh, cc, u0, u1 = _pre_call(x, We, be2, W2, b12, g2, bt2, norm_factor)
    for _ in range(2):
        agg0, agg1 = _spmm_call(srcp, dstp, u0, u1, zeros)
        u0, u1 = _mid_call(agg0, agg1, u0, u1, cc, norm_factor,
                           W1a, W1b, g2, bt2)
    agg0, agg1 = _spmm_call(srcp, dstp, u0, u1, zeros)
    out, = _last_call(agg0, agg1, u0, u1, cc, norm_factor, xh,
                      W1a, W1b, g2, bt2)
    return out


# speedup vs baseline: 1.4215x; 1.1500x over previous
import functools

import jax
import jax.numpy as jnp
from jax import lax
from jax.experimental import pallas as pl
from jax.experimental.pallas import tpu as pltpu
from jax.experimental.pallas import tpu_sc as plsc

_N = 10000
_E = 320000
_DIN = 128
_DH = 256
_H = 128
_ALPHA = 0.5

_NS = 16
_C = 96
_CH = 212
_EPAD = _NS * _CH * _C
_NPAD = 10112
_RPT = _NPAD // _NS
_LAST = _N - (_NS - 1) * _RPT

_R = 1000


def _ln_tanh(v, gamma, beta):
    mu = jnp.mean(v, axis=-1, keepdims=True)
    d = v - mu
    var = jnp.mean(d * d, axis=-1, keepdims=True)
    return jnp.tanh(d * lax.rsqrt(var + 1e-5) * gamma + beta)


def _pre_body(x, We, be, W2, b1, gamma, beta, nf, xh_o, cc_o, u0_o, u1_o):
    xh = jnp.dot(x[...], We[...], preferred_element_type=jnp.float32) + be[...]
    cc = jnp.dot(xh, W2[...], preferred_element_type=jnp.float32) + b1[...]
    h = _ln_tanh(cc, gamma[...], beta[...])
    u = (_ALPHA * h) * nf[...]
    xh_o[...] = xh
    cc_o[...] = cc
    u0_o[...] = u[:, :_H]
    u1_o[...] = u[:, _H:]


def _step_h(agg0, agg1, cc, nfv, W1a, W1b, gamma, beta):
    a0 = agg0[...] * nfv
    a1 = agg1[...] * nfv
    hin = (jnp.dot(a0, W1a[...], preferred_element_type=jnp.float32)
           + jnp.dot(a1, W1b[...], preferred_element_type=jnp.float32)
           + cc[...])
    return _ln_tanh(hin, gamma[...], beta[...])


def _mid_body(agg0, agg1, u0, u1, cc, nf, W1a, W1b, gamma, beta, u0_o, u1_o):
    nfv = nf[...]
    h = _step_h(agg0, agg1, cc, nfv, W1a, W1b, gamma, beta)
    u0_o[...] = (1.0 - _ALPHA) * u0[...] + (_ALPHA * nfv) * h[:, :_H]
    u1_o[...] = (1.0 - _ALPHA) * u1[...] + (_ALPHA * nfv) * h[:, _H:]


def _last_body(agg0, agg1, u0, u1, cc, nf, xh, W1a, W1b, gamma, beta, out_o):
    nfv = nf[...]
    h = _step_h(agg0, agg1, cc, nfv, W1a, W1b, gamma, beta)
    u4 = ((1.0 - _ALPHA) * jnp.concatenate([u0[...], u1[...]], axis=1)
          + (_ALPHA * nfv) * h)
    out_o[...] = u4 + xh[...]


def _row_spec(cols):
    return pl.BlockSpec((_R, cols), lambda i: (i, 0))


def _const_spec(shape):
    return pl.BlockSpec(shape, lambda i: (0, 0))


_GRID = (_N // _R,)
_f32 = jnp.float32

_pre_call = pl.pallas_call(
    _pre_body,
    grid=_GRID,
    in_specs=[
        _row_spec(_DIN),
        _const_spec((_DIN, _DH)),
        _const_spec((1, _DH)),
        _const_spec((_DH, _DH)),
        _const_spec((1, _DH)),
        _const_spec((1, _DH)),
        _const_spec((1, _DH)),
        _row_spec(1),
    ],
    out_specs=[_row_spec(_DH), _row_spec(_DH), _row_spec(_H), _row_spec(_H)],
    out_shape=[jax.ShapeDtypeStruct((_N, _DH), _f32),
               jax.ShapeDtypeStruct((_N, _DH), _f32),
               jax.ShapeDtypeStruct((_N, _H), _f32),
               jax.ShapeDtypeStruct((_N, _H), _f32)],
)

_mid_specs = [
    _row_spec(_H),
    _row_spec(_H),
    _row_spec(_H),
    _row_spec(_H),
    _row_spec(_DH),
    _row_spec(1),
    _const_spec((_H, _DH)),
    _const_spec((_H, _DH)),
    _const_spec((1, _DH)),
    _const_spec((1, _DH)),
]

_mid_call = pl.pallas_call(
    _mid_body,
    grid=_GRID,
    in_specs=_mid_specs,
    out_specs=[_row_spec(_H), _row_spec(_H)],
    out_shape=[jax.ShapeDtypeStruct((_N, _H), _f32),
               jax.ShapeDtypeStruct((_N, _H), _f32)],
)

_last_call = pl.pallas_call(
    _last_body,
    grid=_GRID,
    in_specs=_mid_specs[:6] + [_row_spec(_DH)] + _mid_specs[6:],
    out_specs=[_row_spec(_DH)],
    out_shape=[jax.ShapeDtypeStruct((_N, _DH), _f32)],
)


def _spmm_body(srcp, dstp, u0, u1, zeros, out0, out1,
               src_idx, dst_idx, buf, acc, gsem, isem, *ssem):
    c = lax.axis_index("c")
    s = lax.axis_index("s")

    def run(u_hbm):
        def src_fetch(j, slot):
            pltpu.async_copy(srcp.at[s, j], src_idx.at[slot], isem)

        def dst_fetch(j, slot):
            pltpu.async_copy(dstp.at[s, j], dst_idx.at[slot], isem)

        def src_wait(slot):
            pltpu.make_async_copy(srcp.at[s, 0], src_idx.at[slot], isem).wait()

        def dst_wait(slot):
            pltpu.make_async_copy(dstp.at[s, 0], dst_idx.at[slot], isem).wait()

        def gat_start(slot):
            pltpu.async_copy(u_hbm.at[src_idx.at[slot]], buf.at[slot], gsem)

        def gat_wait(slot):
            pltpu.make_async_copy(u_hbm.at[pl.ds(0, _C)], buf.at[slot],
                                  gsem).wait()

        def scat_start(slot, par):
            pltpu.async_copy(buf.at[slot], acc.at[dst_idx.at[slot]],
                             ssem[par], add=True)

        def scat_wait(par):
            pltpu.make_async_copy(buf.at[0], acc.at[pl.ds(0, _C)],
                                  ssem[par]).wait()

        pltpu.sync_copy(srcp.at[s, 0], src_idx.at[0])
        pltpu.sync_copy(srcp.at[s, 1], src_idx.at[1])
        gat_start(0)
        gat_start(1)
        src_fetch(2, 2)
        dst_fetch(0, 0)
        pltpu.sync_copy(zeros, acc.at[pl.ds(s * _RPT, _RPT)])
        plsc.subcore_barrier()

        def quad(q, carry):
            j = 4 * q
            for r in range(4):
                jr = j + r
                k = r
                pl.when(jr >= 2)(lambda par=r % 2: scat_wait(par))
                pl.when(jr + 3 < _CH)(
                    lambda jr=jr, sl=(r + 3) % 4: src_fetch(jr + 3, sl))
                pl.when(jr + 1 < _CH)(
                    lambda jr=jr, sl=(r + 1) % 4: dst_fetch(jr + 1, sl))
                gat_wait(k)

                def nxt(sl=(r + 2) % 4):
                    src_wait(sl)
                    gat_start(sl)
                pl.when(jr + 2 < _CH)(nxt)
                dst_wait(k)
                scat_start(k, r % 2)
            return carry

        lax.fori_loop(0, _CH // 4, quad, 0)
        scat_wait(0)
        scat_wait(1)

    pl.when(c == 0)(lambda: run(u0))
    pl.when(c == 1)(lambda: run(u1))
    plsc.subcore_barrier()

    def drain(out_hbm):
        def full():
            pltpu.sync_copy(acc.at[pl.ds(s * _RPT, _RPT)],
                            out_hbm.at[pl.ds(s * _RPT, _RPT)])
        def part():
            pltpu.sync_copy(acc.at[pl.ds(s * _RPT, _LAST)],
                            out_hbm.at[pl.ds(s * _RPT, _LAST)])
        pl.when(s < _NS - 1)(full)
        pl.when(s == _NS - 1)(part)

    pl.when(c == 0)(lambda: drain(out0))
    pl.when(c == 1)(lambda: drain(out1))


_spmm_call = pl.kernel(
    _spmm_body,
    out_type=[jax.ShapeDtypeStruct((_N, _H), _f32),
              jax.ShapeDtypeStruct((_N, _H), _f32)],
    mesh=plsc.VectorSubcoreMesh(core_axis_name="c", subcore_axis_name="s"),
    scratch_types=[
        pltpu.VMEM((4, _C), jnp.int32),
        pltpu.VMEM((4, _C), jnp.int32),
        pltpu.VMEM((4, _C, _H), _f32),
        pltpu.VMEM_SHARED((_NPAD, _H), _f32),
        pltpu.SemaphoreType.DMA,
        pltpu.SemaphoreType.DMA,
        pltpu.SemaphoreType.DMA,
        pltpu.SemaphoreType.DMA,
    ],
)


def kernel(x, edge_index, norm_factor, We, be, W1, W2, b1, gamma, beta):
    src = edge_index[0]
    dst = edge_index[1]
    pad = _EPAD - _E
    pad_src = (jnp.arange(pad, dtype=jnp.int32) * 2789) % _N
    pad_dst = _N + (jnp.arange(pad, dtype=jnp.int32) % (_NPAD - _N))
    srcp = jnp.concatenate([src, pad_src]).reshape(_NS, _CH, _C)
    dstp = jnp.concatenate([dst, pad_dst]).reshape(_NS, _CH, _C)
    zeros = jnp.zeros((_RPT, _H), _f32)

    be2 = be.reshape(1, _DH)
    b12 = b1.reshape(1, _DH)
    g2 = gamma.reshape(1, _DH)
    bt2 = beta.reshape(1, _DH)
    W1a = W1[:_H]
    W1b = W1[_H:]

    xh, cc, u0, u1 = _pre_call(x, We, be2, W2, b12, g2, bt2, norm_factor)
    for _ in range(2):
        agg0, agg1 = _spmm_call(srcp, dstp, u0, u1, zeros)
        u0, u1 = _mid_call(agg0, agg1, u0, u1, cc, norm_factor,
                           W1a, W1b, g2, bt2)
    agg0, agg1 = _spmm_call(srcp, dstp, u0, u1, zeros)
    out, = _last_call(agg0, agg1, u0, u1, cc, norm_factor, xh,
                      W1a, W1b, g2, bt2)
    return out


# speedup vs baseline: 1.4575x; 1.0253x over previous
import functools

import jax
import jax.numpy as jnp
from jax import lax
from jax.experimental import pallas as pl
from jax.experimental.pallas import tpu as pltpu
from jax.experimental.pallas import tpu_sc as plsc

_N = 10000
_E = 320000
_DIN = 128
_DH = 256
_H = 128
_ALPHA = 0.5

_NS = 16
_C = 96
_CH = 212
_EPAD = _NS * _CH * _C
_NPAD = 10112
_RPT = _NPAD // _NS
_LAST = _N - (_NS - 1) * _RPT

_R = 2000


def _ln_tanh(v, gamma, beta):
    mu = jnp.mean(v, axis=-1, keepdims=True)
    d = v - mu
    var = jnp.mean(d * d, axis=-1, keepdims=True)
    return jnp.tanh(d * lax.rsqrt(var + 1e-5) * gamma + beta)


def _pre_body(x, We, be, W2, b1, gamma, beta, nf, xh_o, cc_o, u0_o, u1_o):
    xh = jnp.dot(x[...], We[...], preferred_element_type=jnp.float32) + be[...]
    cc = jnp.dot(xh, W2[...], preferred_element_type=jnp.float32) + b1[...]
    h = _ln_tanh(cc, gamma[...], beta[...])
    u = (_ALPHA * h) * nf[...]
    xh_o[...] = xh
    cc_o[...] = cc
    u0_o[...] = u[:, :_H]
    u1_o[...] = u[:, _H:]


def _step_h(agg0, agg1, cc, nfv, W1a, W1b, gamma, beta):
    a0 = agg0[...] * nfv
    a1 = agg1[...] * nfv
    hin = (jnp.dot(a0, W1a[...], preferred_element_type=jnp.float32)
           + jnp.dot(a1, W1b[...], preferred_element_type=jnp.float32)
           + cc[...])
    return _ln_tanh(hin, gamma[...], beta[...])


def _mid_body(agg0, agg1, u0, u1, cc, nf, W1a, W1b, gamma, beta, u0_o, u1_o):
    nfv = nf[...]
    h = _step_h(agg0, agg1, cc, nfv, W1a, W1b, gamma, beta)
    u0_o[...] = (1.0 - _ALPHA) * u0[...] + (_ALPHA * nfv) * h[:, :_H]
    u1_o[...] = (1.0 - _ALPHA) * u1[...] + (_ALPHA * nfv) * h[:, _H:]


def _last_body(agg0, agg1, u0, u1, cc, nf, xh, W1a, W1b, gamma, beta, out_o):
    nfv = nf[...]
    h = _step_h(agg0, agg1, cc, nfv, W1a, W1b, gamma, beta)
    u4 = ((1.0 - _ALPHA) * jnp.concatenate([u0[...], u1[...]], axis=1)
          + (_ALPHA * nfv) * h)
    out_o[...] = u4 + xh[...]


def _row_spec(cols):
    return pl.BlockSpec((_R, cols), lambda i: (i, 0))


def _const_spec(shape):
    return pl.BlockSpec(shape, lambda i: (0, 0))


_GRID = (_N // _R,)
_f32 = jnp.float32

_pre_call = pl.pallas_call(
    _pre_body,
    grid=_GRID,
    in_specs=[
        _row_spec(_DIN),
        _const_spec((_DIN, _DH)),
        _const_spec((1, _DH)),
        _const_spec((_DH, _DH)),
        _const_spec((1, _DH)),
        _const_spec((1, _DH)),
        _const_spec((1, _DH)),
        _row_spec(1),
    ],
    out_specs=[_row_spec(_DH), _row_spec(_DH), _row_spec(_H), _row_spec(_H)],
    out_shape=[jax.ShapeDtypeStruct((_N, _DH), _f32),
               jax.ShapeDtypeStruct((_N, _DH), _f32),
               jax.ShapeDtypeStruct((_N, _H), _f32),
               jax.ShapeDtypeStruct((_N, _H), _f32)],
)

_mid_specs = [
    _row_spec(_H),
    _row_spec(_H),
    _row_spec(_H),
    _row_spec(_H),
    _row_spec(_DH),
    _row_spec(1),
    _const_spec((_H, _DH)),
    _const_spec((_H, _DH)),
    _const_spec((1, _DH)),
    _const_spec((1, _DH)),
]

_mid_call = pl.pallas_call(
    _mid_body,
    grid=_GRID,
    in_specs=_mid_specs,
    out_specs=[_row_spec(_H), _row_spec(_H)],
    out_shape=[jax.ShapeDtypeStruct((_N, _H), _f32),
               jax.ShapeDtypeStruct((_N, _H), _f32)],
)

_last_call = pl.pallas_call(
    _last_body,
    grid=_GRID,
    in_specs=_mid_specs[:6] + [_row_spec(_DH)] + _mid_specs[6:],
    out_specs=[_row_spec(_DH)],
    out_shape=[jax.ShapeDtypeStruct((_N, _DH), _f32)],
)


def _spmm_body(srcp, dstp, u0, u1, zeros, out0, out1,
               src_idx, dst_idx, buf, acc, gsem, isem, *ssem):
    c = lax.axis_index("c")
    s = lax.axis_index("s")

    def run(u_hbm):
        def src_fetch(j, slot):
            pltpu.async_copy(srcp.at[s, j], src_idx.at[slot], isem)

        def dst_fetch(j, slot):
            pltpu.async_copy(dstp.at[s, j], dst_idx.at[slot], isem)

        def src_wait(slot):
            pltpu.make_async_copy(srcp.at[s, 0], src_idx.at[slot], isem).wait()

        def dst_wait(slot):
            pltpu.make_async_copy(dstp.at[s, 0], dst_idx.at[slot], isem).wait()

        def gat_start(slot):
            pltpu.async_copy(u_hbm.at[src_idx.at[slot]], buf.at[slot], gsem)

        def gat_wait(slot):
            pltpu.make_async_copy(u_hbm.at[pl.ds(0, _C)], buf.at[slot],
                                  gsem).wait()

        def scat_start(slot, par):
            pltpu.async_copy(buf.at[slot], acc.at[dst_idx.at[slot]],
                             ssem[par], add=True)

        def scat_wait(par):
            pltpu.make_async_copy(buf.at[0], acc.at[pl.ds(0, _C)],
                                  ssem[par]).wait()

        pltpu.sync_copy(srcp.at[s, 0], src_idx.at[0])
        pltpu.sync_copy(srcp.at[s, 1], src_idx.at[1])
        gat_start(0)
        gat_start(1)
        src_fetch(2, 2)
        dst_fetch(0, 0)
        pltpu.sync_copy(zeros, acc.at[pl.ds(s * _RPT, _RPT)])
        plsc.subcore_barrier()

        def quad(q, carry):
            j = 4 * q
            for r in range(4):
                jr = j + r
                k = r
                pl.when(jr >= 2)(lambda par=r % 2: scat_wait(par))
                pl.when(jr + 3 < _CH)(
                    lambda jr=jr, sl=(r + 3) % 4: src_fetch(jr + 3, sl))
                pl.when(jr + 1 < _CH)(
                    lambda jr=jr, sl=(r + 1) % 4: dst_fetch(jr + 1, sl))
                gat_wait(k)

                def nxt(sl=(r + 2) % 4):
                    src_wait(sl)
                    gat_start(sl)
                pl.when(jr + 2 < _CH)(nxt)
                dst_wait(k)
                scat_start(k, r % 2)
            return carry

        lax.fori_loop(0, _CH // 4, quad, 0)
        scat_wait(0)
        scat_wait(1)

    pl.when(c == 0)(lambda: run(u0))
    pl.when(c == 1)(lambda: run(u1))
    plsc.subcore_barrier()

    def drain(out_hbm):
        def full():
            pltpu.sync_copy(acc.at[pl.ds(s * _RPT, _RPT)],
                            out_hbm.at[pl.ds(s * _RPT, _RPT)])
        def part():
            pltpu.sync_copy(acc.at[pl.ds(s * _RPT, _LAST)],
                            out_hbm.at[pl.ds(s * _RPT, _LAST)])
        pl.when(s < _NS - 1)(full)
        pl.when(s == _NS - 1)(part)

    pl.when(c == 0)(lambda: drain(out0))
    pl.when(c == 1)(lambda: drain(out1))


_spmm_call = pl.kernel(
    _spmm_body,
    out_type=[jax.ShapeDtypeStruct((_N, _H), _f32),
              jax.ShapeDtypeStruct((_N, _H), _f32)],
    mesh=plsc.VectorSubcoreMesh(core_axis_name="c", subcore_axis_name="s"),
    scratch_types=[
        pltpu.VMEM((4, _C), jnp.int32),
        pltpu.VMEM((4, _C), jnp.int32),
        pltpu.VMEM((4, _C, _H), _f32),
        pltpu.VMEM_SHARED((_NPAD, _H), _f32),
        pltpu.SemaphoreType.DMA,
        pltpu.SemaphoreType.DMA,
        pltpu.SemaphoreType.DMA,
        pltpu.SemaphoreType.DMA,
    ],
)


def kernel(x, edge_index, norm_factor, We, be, W1, W2, b1, gamma, beta):
    src = edge_index[0]
    dst = edge_index[1]
    pad = _EPAD - _E
    pad_src = (jnp.arange(pad, dtype=jnp.int32) * 2789) % _N
    pad_dst = _N + (jnp.arange(pad, dtype=jnp.int32) % (_NPAD - _N))
    srcp = jnp.concatenate([src, pad_src]).reshape(_NS, _CH, _C)
    dstp = jnp.concatenate([dst, pad_dst]).reshape(_NS, _CH, _C)
    zeros = jnp.zeros((_RPT, _H), _f32)

    be2 = be.reshape(1, _DH)
    b12 = b1.reshape(1, _DH)
    g2 = gamma.reshape(1, _DH)
    bt2 = beta.reshape(1, _DH)
    W1a = W1[:_H]
    W1b = W1[_H:]

    xh, cc, u0, u1 = _pre_call(x, We, be2, W2, b12, g2, bt2, norm_factor)
    for _ in range(2):
        agg0, agg1 = _spmm_call(srcp, dstp, u0, u1, zeros)
        u0, u1 = _mid_call(agg0, agg1, u0, u1, cc, norm_factor,
                           W1a, W1b, g2, bt2)
    agg0, agg1 = _spmm_call(srcp, dstp, u0, u1, zeros)
    out, = _last_call(agg0, agg1, u0, u1, cc, norm_factor, xh,
                      W1a, W1b, g2, bt2)
    return out


# speedup vs baseline: 1.4611x; 1.0025x over previous
import functools

import jax
import jax.numpy as jnp
from jax import lax
from jax.experimental import pallas as pl
from jax.experimental.pallas import tpu as pltpu
from jax.experimental.pallas import tpu_sc as plsc

_N = 10000
_E = 320000
_DIN = 128
_DH = 256
_H = 128
_ALPHA = 0.5

_NS = 16
_C = 96
_CH = 212
_EPAD = _NS * _CH * _C
_NPAD = 10112
_RPT = _NPAD // _NS
_LAST = _N - (_NS - 1) * _RPT

_R = 2000


def _ln_tanh(v, gamma, beta):
    mu = jnp.mean(v, axis=-1, keepdims=True)
    d = v - mu
    var = jnp.mean(d * d, axis=-1, keepdims=True)
    return jnp.tanh(d * lax.rsqrt(var + 1e-5) * gamma + beta)


def _pre_body(x, We, be, W2, b1, gamma, beta, nf, xh_o, cc_o, u0_o, u1_o):
    xh = jnp.dot(x[...], We[...], preferred_element_type=jnp.float32) + be[...]
    cc = jnp.dot(xh, W2[...], preferred_element_type=jnp.float32) + b1[...]
    h = _ln_tanh(cc, gamma[...], beta[...])
    u = (_ALPHA * h) * nf[...]
    xh_o[...] = xh
    cc_o[...] = cc
    u0_o[...] = u[:, :_H]
    u1_o[...] = u[:, _H:]


def _step_h(agg0, agg1, cc, nfv, W1a, W1b, gamma, beta):
    a0 = agg0[...] * nfv
    a1 = agg1[...] * nfv
    hin = (jnp.dot(a0, W1a[...], preferred_element_type=jnp.float32)
           + jnp.dot(a1, W1b[...], preferred_element_type=jnp.float32)
           + cc[...])
    return _ln_tanh(hin, gamma[...], beta[...])


def _mid_body(agg0, agg1, u0, u1, cc, nf, W1a, W1b, gamma, beta, u0_o, u1_o):
    nfv = nf[...]
    h = _step_h(agg0, agg1, cc, nfv, W1a, W1b, gamma, beta)
    u0_o[...] = (1.0 - _ALPHA) * u0[...] + (_ALPHA * nfv) * h[:, :_H]
    u1_o[...] = (1.0 - _ALPHA) * u1[...] + (_ALPHA * nfv) * h[:, _H:]


def _last_body(agg0, agg1, u0, u1, cc, nf, xh, W1a, W1b, gamma, beta, out_o):
    nfv = nf[...]
    h = _step_h(agg0, agg1, cc, nfv, W1a, W1b, gamma, beta)
    xhv = xh[...]
    out_o[:, :_H] = ((1.0 - _ALPHA) * u0[...] + (_ALPHA * nfv) * h[:, :_H]
                     + xhv[:, :_H])
    out_o[:, _H:] = ((1.0 - _ALPHA) * u1[...] + (_ALPHA * nfv) * h[:, _H:]
                     + xhv[:, _H:])


def _row_spec(cols):
    return pl.BlockSpec((_R, cols), lambda i: (i, 0))


def _const_spec(shape):
    return pl.BlockSpec(shape, lambda i: (0, 0))


_GRID = (_N // _R,)
_f32 = jnp.float32

_pre_call = pl.pallas_call(
    _pre_body,
    grid=_GRID,
    in_specs=[
        _row_spec(_DIN),
        _const_spec((_DIN, _DH)),
        _const_spec((1, _DH)),
        _const_spec((_DH, _DH)),
        _const_spec((1, _DH)),
        _const_spec((1, _DH)),
        _const_spec((1, _DH)),
        _row_spec(1),
    ],
    out_specs=[_row_spec(_DH), _row_spec(_DH), _row_spec(_H), _row_spec(_H)],
    out_shape=[jax.ShapeDtypeStruct((_N, _DH), _f32),
               jax.ShapeDtypeStruct((_N, _DH), _f32),
               jax.ShapeDtypeStruct((_N, _H), _f32),
               jax.ShapeDtypeStruct((_N, _H), _f32)],
)

_mid_specs = [
    _row_spec(_H),
    _row_spec(_H),
    _row_spec(_H),
    _row_spec(_H),
    _row_spec(_DH),
    _row_spec(1),
    _const_spec((_H, _DH)),
    _const_spec((_H, _DH)),
    _const_spec((1, _DH)),
    _const_spec((1, _DH)),
]

_mid_call = pl.pallas_call(
    _mid_body,
    grid=_GRID,
    in_specs=_mid_specs,
    out_specs=[_row_spec(_H), _row_spec(_H)],
    out_shape=[jax.ShapeDtypeStruct((_N, _H), _f32),
               jax.ShapeDtypeStruct((_N, _H), _f32)],
)

_last_call = pl.pallas_call(
    _last_body,
    grid=_GRID,
    in_specs=_mid_specs[:6] + [_row_spec(_DH)] + _mid_specs[6:],
    out_specs=[_row_spec(_DH)],
    out_shape=[jax.ShapeDtypeStruct((_N, _DH), _f32)],
)


def _spmm_body(srcp, dstp, u0, u1, zeros, out0, out1,
               src_idx, dst_idx, buf, acc, gsem, isem, *ssem):
    c = lax.axis_index("c")
    s = lax.axis_index("s")

    def run(u_hbm):
        def src_fetch(j, slot):
            pltpu.async_copy(srcp.at[s, j], src_idx.at[slot], isem)

        def dst_fetch(j, slot):
            pltpu.async_copy(dstp.at[s, j], dst_idx.at[slot], isem)

        def src_wait(slot):
            pltpu.make_async_copy(srcp.at[s, 0], src_idx.at[slot], isem).wait()

        def dst_wait(slot):
            pltpu.make_async_copy(dstp.at[s, 0], dst_idx.at[slot], isem).wait()

        def gat_start(slot):
            pltpu.async_copy(u_hbm.at[src_idx.at[slot]], buf.at[slot], gsem)

        def gat_wait(slot):
            pltpu.make_async_copy(u_hbm.at[pl.ds(0, _C)], buf.at[slot],
                                  gsem).wait()

        def scat_start(slot, par):
            pltpu.async_copy(buf.at[slot], acc.at[dst_idx.at[slot]],
                             ssem[par], add=True)

        def scat_wait(par):
            pltpu.make_async_copy(buf.at[0], acc.at[pl.ds(0, _C)],
                                  ssem[par]).wait()

        pltpu.sync_copy(srcp.at[s, 0], src_idx.at[0])
        pltpu.sync_copy(srcp.at[s, 1], src_idx.at[1])
        gat_start(0)
        gat_start(1)
        src_fetch(2, 2)
        dst_fetch(0, 0)
        pltpu.sync_copy(zeros, acc.at[pl.ds(s * _RPT, _RPT)])
        plsc.subcore_barrier()

        def quad(q, carry):
            j = 4 * q
            for r in range(4):
                jr = j + r
                k = r
                pl.when(jr >= 2)(lambda par=r % 2: scat_wait(par))
                pl.when(jr + 3 < _CH)(
                    lambda jr=jr, sl=(r + 3) % 4: src_fetch(jr + 3, sl))
                pl.when(jr + 1 < _CH)(
                    lambda jr=jr, sl=(r + 1) % 4: dst_fetch(jr + 1, sl))
                gat_wait(k)

                def nxt(sl=(r + 2) % 4):
                    src_wait(sl)
                    gat_start(sl)
                pl.when(jr + 2 < _CH)(nxt)
                dst_wait(k)
                scat_start(k, r % 2)
            return carry

        lax.fori_loop(0, _CH // 4, quad, 0)
        scat_wait(0)
        scat_wait(1)

    pl.when(c == 0)(lambda: run(u0))
    pl.when(c == 1)(lambda: run(u1))
    plsc.subcore_barrier()

    def drain(out_hbm):
        def full():
            pltpu.sync_copy(acc.at[pl.ds(s * _RPT, _RPT)],
                            out_hbm.at[pl.ds(s * _RPT, _RPT)])
        def part():
            pltpu.sync_copy(acc.at[pl.ds(s * _RPT, _LAST)],
                            out_hbm.at[pl.ds(s * _RPT, _LAST)])
        pl.when(s < _NS - 1)(full)
        pl.when(s == _NS - 1)(part)

    pl.when(c == 0)(lambda: drain(out0))
    pl.when(c == 1)(lambda: drain(out1))


_spmm_call = pl.kernel(
    _spmm_body,
    out_type=[jax.ShapeDtypeStruct((_N, _H), _f32),
              jax.ShapeDtypeStruct((_N, _H), _f32)],
    mesh=plsc.VectorSubcoreMesh(core_axis_name="c", subcore_axis_name="s"),
    scratch_types=[
        pltpu.VMEM((4, _C), jnp.int32),
        pltpu.VMEM((4, _C), jnp.int32),
        pltpu.VMEM((4, _C, _H), _f32),
        pltpu.VMEM_SHARED((_NPAD, _H), _f32),
        pltpu.SemaphoreType.DMA,
        pltpu.SemaphoreType.DMA,
        pltpu.SemaphoreType.DMA,
        pltpu.SemaphoreType.DMA,
    ],
)


def kernel(x, edge_index, norm_factor, We, be, W1, W2, b1, gamma, beta):
    src = edge_index[0]
    dst = edge_index[1]
    pad = _EPAD - _E
    pad_src = (jnp.arange(pad, dtype=jnp.int32) * 2789) % _N
    pad_dst = _N + (jnp.arange(pad, dtype=jnp.int32) % (_NPAD - _N))
    srcp = jnp.concatenate([src, pad_src]).reshape(_NS, _CH, _C)
    dstp = jnp.concatenate([dst, pad_dst]).reshape(_NS, _CH, _C)
    zeros = jnp.zeros((_RPT, _H), _f32)

    be2 = be.reshape(1, _DH)
    b12 = b1.reshape(1, _DH)
    g2 = gamma.reshape(1, _DH)
    bt2 = beta.reshape(1, _DH)
    W1a = W1[:_H]
    W1b = W1[_H:]

    xh, cc, u0, u1 = _pre_call(x, We, be2, W2, b12, g2, bt2, norm_factor)
    for _ in range(2):
        agg0, agg1 = _spmm_call(srcp, dstp, u0, u1, zeros)
        u0, u1 = _mid_call(agg0, agg1, u0, u1, cc, norm_factor,
                           W1a, W1b, g2, bt2)
    agg0, agg1 = _spmm_call(srcp, dstp, u0, u1, zeros)
    out, = _last_call(agg0, agg1, u0, u1, cc, norm_factor, xh,
                      W1a, W1b, g2, bt2)
    return out
